# baseline (speedup 1.0000x reference)
import functools

import jax
import jax.numpy as jnp
from jax import lax
from jax.experimental import pallas as pl
from jax.experimental.pallas import tpu as pltpu
from jax.experimental.pallas import tpu_sc as plsc

B = 16384
D = 128
NC = 2
NS = 16
NW = NC * NS
CH = 8
NBUF = 3

B_SC = 8192
B_TC = B - B_SC
RB_G = 16


def _build_sc(b_sc):
    b_per_w = b_sc // NW
    rpc = b_per_w // CH
    mesh = plsc.VectorSubcoreMesh(core_axis_name="c", subcore_axis_name="s")

    @functools.partial(
        pl.kernel,
        mesh=mesh,
        out_type=jax.ShapeDtypeStruct((b_sc, D), jnp.float32),
        scratch_types=[
            pltpu.VMEM((CH, rpc), jnp.int32),
            pltpu.VMEM_SHARED((NS, 4, D), jnp.float32),
            pltpu.VMEM((NBUF, rpc, D), jnp.float32),
            pltpu.SemaphoreType.DMA,
            pltpu.SemaphoreType.DMA,
            pltpu.SemaphoreType.DMA,
        ],
    )
    def gather_kernel(idx_hbm, table_hbm, out_hbm, idx_v, tbl_sh, rows_v,
                      ssem, gsem, osem):
        sid = lax.axis_index("s")
        wid = sid * NC + lax.axis_index("c")
        base = wid * b_per_w
        ic = pltpu.async_copy(idx_hbm.at[pl.ds(wid * CH, CH)], idx_v, ssem)
        tcopy = pltpu.async_copy(table_hbm, tbl_sh.at[sid], ssem)
        ic.wait()
        tcopy.wait()
        tbl = tbl_sh.at[sid]

        gs = [None] * CH
        outs = [None] * CH
        for c in range(min(NBUF, CH)):
            gs[c] = pltpu.async_copy(
                tbl.at[idx_v.at[c]], rows_v.at[c % NBUF], gsem)
        for c in range(CH):
            gs[c].wait()
            outs[c] = pltpu.async_copy(
                rows_v.at[c % NBUF],
                out_hbm.at[pl.ds(base + c * rpc, rpc)], osem)
            n = c + NBUF - 1
            if NBUF <= n < CH and gs[n] is None:
                outs[n - NBUF].wait()
                gs[n] = pltpu.async_copy(
                    tbl.at[idx_v.at[n]], rows_v.at[n % NBUF], gsem)
        for c in range(max(0, CH - NBUF), CH):
            outs[c].wait()

    return gather_kernel


def _tc_body(idx_ref, tbl_ref, out_ref):
    t = tbl_ref[...]
    kiota = lax.broadcasted_iota(jnp.int32, (4, 128), 0)
    for g in range(RB_G):
        row = idx_ref[g:g + 1, :]
        oh = (row == kiota).astype(jnp.float32)
        blk = lax.dot_general(oh, t, (((0,), (0,)), ((), ())),
                              precision=lax.Precision.HIGHEST,
                              preferred_element_type=jnp.float32)
        out_ref[pl.ds(g * 128, 128), :] = blk


def _tc_select(idx2d, tbl):
    ntb = B_TC // (RB_G * 128)
    return pl.pallas_call(
        _tc_body,
        grid=(ntb,),
        in_specs=[
            pl.BlockSpec((RB_G, 128), lambda i: (i, 0)),
            pl.BlockSpec((4, D), lambda i: (0, 0)),
        ],
        out_specs=pl.BlockSpec((RB_G * 128, D), lambda i: (i, 0)),
        out_shape=jax.ShapeDtypeStruct((B_TC, D), jnp.float32),
    )(idx2d, tbl)


_GATHER = None


def kernel(emotion_label, table):
    global _GATHER
    if _GATHER is None:
        _GATHER = _build_sc(B_SC)
    idx = emotion_label.astype(jnp.int32)
    rpc = B_SC // NW // CH
    sc_out = _GATHER(idx[:B_SC].reshape(NW * CH, rpc), table)
    tc_out = _tc_select(idx[B_SC:].reshape(-1, 128), table)
    return jnp.concatenate([sc_out, tc_out], axis=0)


# speedup vs baseline: 1.2650x; 1.2650x over previous
import functools

import jax
import jax.numpy as jnp
from jax import lax
from jax.experimental import pallas as pl
from jax.experimental.pallas import tpu as pltpu
from jax.experimental.pallas import tpu_sc as plsc

B = 16384
D = 128
NC = 2
NS = 16
NW = NC * NS
B_PER_W = B // NW
CH = 4
RPC = B_PER_W // CH


def _build():
    mesh = plsc.VectorSubcoreMesh(core_axis_name="c", subcore_axis_name="s")

    @functools.partial(
        pl.kernel,
        mesh=mesh,
        out_type=jax.ShapeDtypeStruct((B, D), jnp.float32),
        scratch_types=[
            pltpu.VMEM((CH, RPC), jnp.int32),
            pltpu.VMEM_SHARED((NS, 4, D), jnp.float32),
            pltpu.VMEM((CH, RPC, D), jnp.float32),
            pltpu.SemaphoreType.DMA,
            pltpu.SemaphoreType.DMA,
            pltpu.SemaphoreType.DMA,
        ],
    )
    def gather_kernel(idx_hbm, table_hbm, out_hbm, idx_v, tbl_sh, rows_v,
                      ssem, gsem, osem):
        sid = lax.axis_index("s")
        wid = sid * NC + lax.axis_index("c")
        base = wid * B_PER_W
        ic = pltpu.async_copy(idx_hbm.at[pl.ds(wid * CH, CH)], idx_v, ssem)
        tcopy = pltpu.async_copy(table_hbm, tbl_sh.at[sid], ssem)
        ic.wait()
        tcopy.wait()
        tbl = tbl_sh.at[sid]

        gs = [
            pltpu.async_copy(tbl.at[idx_v.at[c]], rows_v.at[c], gsem)
            for c in range(CH)
        ]
        outs = []
        for c in range(CH):
            gs[c].wait()
            outs.append(pltpu.async_copy(
                rows_v.at[c],
                out_hbm.at[pl.ds(base + c * RPC, RPC)], osem))
        for o in outs:
            o.wait()

    return gather_kernel


_GATHER = None


def kernel(emotion_label, table):
    global _GATHER
    if _GATHER is None:
        _GATHER = _build()
    idx = emotion_label.astype(jnp.int32).reshape(NW * CH, RPC)
    return _GATHER(idx, table)


# speedup vs baseline: 1.2940x; 1.0230x over previous
import functools

import jax
import jax.numpy as jnp
from jax import lax
from jax.experimental import pallas as pl
from jax.experimental.pallas import tpu as pltpu
from jax.experimental.pallas import tpu_sc as plsc

B = 16384
D = 128
NC = 2
NS = 16
NW = NC * NS
B_PER_W = B // NW
CH = 4
RPC = B_PER_W // CH


def _build():
    mesh = plsc.VectorSubcoreMesh(core_axis_name="c", subcore_axis_name="s")

    @functools.partial(
        pl.kernel,
        mesh=mesh,
        out_type=jax.ShapeDtypeStruct((B, D), jnp.float32),
        scratch_types=[
            pltpu.VMEM((CH, RPC), jnp.int32),
            pltpu.VMEM_SHARED((4, D), jnp.float32),
            pltpu.VMEM((CH, RPC, D), jnp.float32),
            pltpu.SemaphoreType.DMA,
            pltpu.SemaphoreType.DMA,
            pltpu.SemaphoreType.DMA,
        ],
    )
    def gather_kernel(idx_hbm, table_hbm, out_hbm, idx_v, tbl_sh, rows_v,
                      ssem, gsem, osem):
        sid = lax.axis_index("s")
        wid = sid * NC + lax.axis_index("c")
        base = wid * B_PER_W
        ic = pltpu.async_copy(idx_hbm.at[pl.ds(wid * CH, CH)], idx_v, ssem)

        @pl.when(sid == 0)
        def _():
            pltpu.sync_copy(table_hbm, tbl_sh)

        ic.wait()
        plsc.subcore_barrier()
        tbl = tbl_sh

        gs = [
            pltpu.async_copy(tbl.at[idx_v.at[c]], rows_v.at[c], gsem)
            for c in range(CH)
        ]
        outs = []
        for c in range(CH):
            gs[c].wait()
            outs.append(pltpu.async_copy(
                rows_v.at[c],
                out_hbm.at[pl.ds(base + c * RPC, RPC)], osem))
        for o in outs:
            o.wait()

    return gather_kernel


_GATHER = None


def kernel(emotion_label, table):
    global _GATHER
    if _GATHER is None:
        _GATHER = _build()
    idx = emotion_label.astype(jnp.int32).reshape(NW * CH, RPC)
    return _GATHER(idx, table)
